# Initial kernel scaffold; baseline (speedup 1.0000x reference)
import jax
import jax.numpy as jnp
from jax.experimental import pallas as pl


def kernel(x_game, x_state, edge_index_gg, edge_index_gs, edge_attr, W_self_g, W_nbr_g, b_g, W_edge, W_self_s, W_nbr_s, b_s, W_lin, b_lin):
    raise NotImplementedError("write your pallas kernel here")


# speedup vs baseline: 3.4169x; 3.4169x over previous
import functools

import jax
import jax.numpy as jnp
from jax import lax
from jax.experimental import pallas as pl
from jax.experimental.pallas import tpu as pltpu
from jax.experimental.pallas import tpu_sc as plsc

NG = 10000
NS_NODES = 10000
EGG = 160000
EGS = 160000
D = 256
HID = 512
OUT = 256
DE = 16
DE_PAD = 128

NC, NSUB = 2, 16
CH = 128
NCHUNK = EGG // CH
ROWS_MAIN = 624
ROWS_LAST = NG - 15 * ROWS_MAIN


def _zero_own(w, zsrc, acc):
  @pl.when(w < NSUB - 1)
  def _():
    pltpu.sync_copy(zsrc.at[pl.ds(0, ROWS_MAIN)],
                    acc.at[pl.ds(w * ROWS_MAIN, ROWS_MAIN)])
  @pl.when(w == NSUB - 1)
  def _():
    pltpu.sync_copy(zsrc, acc.at[pl.ds(15 * ROWS_MAIN, ROWS_LAST)])


def _copy_own(w, acc, out):
  @pl.when(w < NSUB - 1)
  def _():
    pltpu.sync_copy(acc.at[pl.ds(w * ROWS_MAIN, ROWS_MAIN)],
                    out.at[pl.ds(w * ROWS_MAIN, ROWS_MAIN)])
  @pl.when(w == NSUB - 1)
  def _():
    pltpu.sync_copy(acc.at[pl.ds(15 * ROWS_MAIN, ROWS_LAST)],
                    out.at[pl.ds(15 * ROWS_MAIN, ROWS_LAST)])

_f32 = jnp.float32


def _edge_pass(idx_src_hbm, idx_dst_hbm, table_hbm, acc_shared,
               src_v, dst_v, rows_v, sem, w):
  def body(i, carry):
    q = i * NSUB + w
    @pl.when(q < NCHUNK)
    def _():
      off = q * CH
      pltpu.sync_copy(idx_src_hbm.at[pl.ds(off, CH)], src_v)
      pltpu.sync_copy(idx_dst_hbm.at[pl.ds(off, CH)], dst_v)
      pltpu.async_copy(table_hbm.at[src_v], rows_v, sem).wait()
      pltpu.sync_copy(rows_v, acc_shared.at[dst_v], add=True)
    return carry
  lax.fori_loop(0, (NCHUNK + NSUB - 1) // NSUB, body, 0)


@functools.cache
def _sc_kernels():
  mesh = plsc.VectorSubcoreMesh(
      core_axis_name="c", subcore_axis_name="s",
      num_cores=NC, num_subcores=NSUB)

  @functools.partial(
      pl.kernel,
      out_type=(
          jax.ShapeDtypeStruct((NG, 128), _f32),
          jax.ShapeDtypeStruct((NG, 128), _f32),
      ),
      mesh=mesh,
      scratch_types=[
          pltpu.VMEM((CH,), jnp.int32),
          pltpu.VMEM((CH,), jnp.int32),
          pltpu.VMEM((CH, 128), _f32),
          pltpu.VMEM_SHARED((NG, 128), _f32),
          pltpu.SemaphoreType.DMA,
      ],
  )
  def _sc_stage1(xg0, xg1, src_gg, dst_gg, zx,
                 agg0, agg1,
                 src_v, dst_v, rows_v, acc_x, sem):
    c = lax.axis_index("c")
    w = lax.axis_index("s")

    _zero_own(w, zx, acc_x)
    plsc.subcore_barrier()

    @pl.when(c == 0)
    def _():
      _edge_pass(src_gg, dst_gg, xg0, acc_x, src_v, dst_v, rows_v, sem, w)
    @pl.when(c == 1)
    def _():
      _edge_pass(src_gg, dst_gg, xg1, acc_x, src_v, dst_v, rows_v, sem, w)

    plsc.subcore_barrier()

    @pl.when(c == 0)
    def _():
      _copy_own(w, acc_x, agg0)
    @pl.when(c == 1)
    def _():
      _copy_own(w, acc_x, agg1)

  @functools.partial(
      pl.kernel,
      out_type=(
          jax.ShapeDtypeStruct((NS_NODES, DE_PAD), _f32),
          jax.ShapeDtypeStruct((NS_NODES, DE_PAD), _f32),
      ),
      mesh=mesh,
      scratch_types=[
          pltpu.VMEM((CH,), jnp.int32),
          pltpu.VMEM((CH, DE_PAD), _f32),
          pltpu.VMEM_SHARED((NS_NODES, DE_PAD), _f32),
          pltpu.SemaphoreType.DMA,
      ],
  )
  def _sc_edges(eext, dst_gs, ze, e0, e1, dst_v, erows_v, acc_e, sem):
    c = lax.axis_index("c")
    w = lax.axis_index("s")

    _zero_own(w, ze, acc_e)
    plsc.subcore_barrier()

    half = NCHUNK // 2
    def ebody(i, carry):
      t = i * NSUB + w
      @pl.when(t < half)
      def _():
        off = (c * half + t) * CH
        pltpu.sync_copy(dst_gs.at[pl.ds(off, CH)], dst_v)
        pltpu.sync_copy(eext.at[pl.ds(off, CH)], erows_v)
        pltpu.sync_copy(erows_v, acc_e.at[dst_v], add=True)
      return carry
    lax.fori_loop(0, (half + NSUB - 1) // NSUB, ebody, 0)

    plsc.subcore_barrier()

    @pl.when(c == 0)
    def _():
      _copy_own(w, acc_e, e0)
    @pl.when(c == 1)
    def _():
      _copy_own(w, acc_e, e1)

  @functools.partial(
      pl.kernel,
      out_type=tuple(
          jax.ShapeDtypeStruct((NS_NODES, 128), _f32) for _ in range(4)),
      mesh=mesh,
      scratch_types=[
          pltpu.VMEM((CH,), jnp.int32),
          pltpu.VMEM((CH,), jnp.int32),
          pltpu.VMEM((CH, 128), _f32),
          pltpu.VMEM_SHARED((NS_NODES, 128), _f32),
          pltpu.SemaphoreType.DMA,
      ],
  )
  def _sc_stage2(h0, h1, h2, h3, src_gs, dst_gs, zx,
                 g0, g1, g2, g3,
                 src_v, dst_v, rows_v, acc, sem):
    c = lax.axis_index("c")
    w = lax.axis_index("s")

    def one_pass(tbl, out):
      _zero_own(w, zx, acc)
      plsc.subcore_barrier()
      _edge_pass(src_gs, dst_gs, tbl, acc, src_v, dst_v, rows_v, sem, w)
      plsc.subcore_barrier()
      _copy_own(w, acc, out)
      plsc.subcore_barrier()

    @pl.when(c == 0)
    def _():
      one_pass(h0, g0)
      one_pass(h1, g1)
    @pl.when(c == 1)
    def _():
      one_pass(h2, g2)
      one_pass(h3, g3)

  return _sc_stage1, _sc_edges, _sc_stage2


_RB = 1000


def _tc1_body(x_ref, a0_ref, a1_ref, ws_ref, wn_ref, b_ref,
              o0, o1, o2, o3):
  h = jnp.dot(x_ref[...], ws_ref[...], preferred_element_type=_f32)
  h += jnp.dot(a0_ref[...], wn_ref[0:128, :], preferred_element_type=_f32)
  h += jnp.dot(a1_ref[...], wn_ref[128:256, :], preferred_element_type=_f32)
  h = jnp.maximum(h + b_ref[...], 0.0)
  o0[...] = h[:, 0:128]
  o1[...] = h[:, 128:256]
  o2[...] = h[:, 256:384]
  o3[...] = h[:, 384:512]


def _tc1(x_game, agg0, agg1, w_self, w_nbr, b):
  grid = (NG // _RB,)
  return pl.pallas_call(
      _tc1_body,
      grid=grid,
      in_specs=[
          pl.BlockSpec((_RB, D), lambda i: (i, 0)),
          pl.BlockSpec((_RB, 128), lambda i: (i, 0)),
          pl.BlockSpec((_RB, 128), lambda i: (i, 0)),
          pl.BlockSpec((D, HID), lambda i: (0, 0)),
          pl.BlockSpec((D, HID), lambda i: (0, 0)),
          pl.BlockSpec((1, HID), lambda i: (0, 0)),
      ],
      out_specs=tuple(
          pl.BlockSpec((_RB, 128), lambda i: (i, 0)) for _ in range(4)),
      out_shape=tuple(
          jax.ShapeDtypeStruct((NG, 128), _f32) for _ in range(4)),
  )(x_game, agg0, agg1, w_self, w_nbr, b)


def _tc2_body(xs_ref, g0, g1, g2, g3, e0, e1, we_ref, wss_ref, wns_ref,
              bs_ref, wl_ref, bl_ref, out_ref):
  ee = e0[...] + e1[...]
  cnt = jnp.maximum(ee[:, DE:DE + 1], 1.0)
  hg = jnp.concatenate([g0[...], g1[...], g2[...], g3[...]], axis=1)
  pre = hg + jnp.dot(ee[:, 0:DE], we_ref[...], preferred_element_type=_f32)
  agg_s = jnp.dot(pre, wns_ref[...], preferred_element_type=_f32) / cnt
  hs = jnp.dot(xs_ref[...], wss_ref[...], preferred_element_type=_f32)
  hs = jnp.maximum(hs + agg_s + bs_ref[...], 0.0)
  out_ref[...] = (
      jnp.dot(hs, wl_ref[...], preferred_element_type=_f32) + bl_ref[...])


def _tc2(x_state, g0, g1, g2, g3, e0, e1, w_edge, w_self_s, w_nbr_s, b_s,
         w_lin, b_lin):
  grid = (NS_NODES // _RB,)
  return pl.pallas_call(
      _tc2_body,
      grid=grid,
      in_specs=[
          pl.BlockSpec((_RB, D), lambda i: (i, 0)),
          pl.BlockSpec((_RB, 128), lambda i: (i, 0)),
          pl.BlockSpec((_RB, 128), lambda i: (i, 0)),
          pl.BlockSpec((_RB, 128), lambda i: (i, 0)),
          pl.BlockSpec((_RB, 128), lambda i: (i, 0)),
          pl.BlockSpec((_RB, DE_PAD), lambda i: (i, 0)),
          pl.BlockSpec((_RB, DE_PAD), lambda i: (i, 0)),
          pl.BlockSpec((DE, HID), lambda i: (0, 0)),
          pl.BlockSpec((D, HID), lambda i: (0, 0)),
          pl.BlockSpec((HID, HID), lambda i: (0, 0)),
          pl.BlockSpec((1, HID), lambda i: (0, 0)),
          pl.BlockSpec((HID, OUT), lambda i: (0, 0)),
          pl.BlockSpec((1, OUT), lambda i: (0, 0)),
      ],
      out_specs=pl.BlockSpec((_RB, OUT), lambda i: (i, 0)),
      out_shape=jax.ShapeDtypeStruct((NS_NODES, OUT), _f32),
  )(x_state, g0, g1, g2, g3, e0, e1, w_edge, w_self_s, w_nbr_s, b_s,
    w_lin, b_lin)


def kernel(x_game, x_state, edge_index_gg, edge_index_gs, edge_attr,
           W_self_g, W_nbr_g, b_g, W_edge, W_self_s, W_nbr_s, b_s,
           W_lin, b_lin):
  xg0 = x_game[:, 0:128]
  xg1 = x_game[:, 128:256]
  src_gg = edge_index_gg[0]
  dst_gg = edge_index_gg[1]
  src_gs = edge_index_gs[0]
  dst_gs = edge_index_gs[1]
  eext = jnp.concatenate(
      [edge_attr,
       jnp.ones((EGS, 1), _f32),
       jnp.zeros((EGS, DE_PAD - DE - 1), _f32)], axis=1)
  zx = jnp.zeros((ROWS_LAST, 128), _f32)
  ze = jnp.zeros((ROWS_LAST, DE_PAD), _f32)

  sc_stage1, sc_edges, sc_stage2 = _sc_kernels()
  agg0, agg1 = sc_stage1(xg0, xg1, src_gg, dst_gg, zx)
  e0, e1 = sc_edges(eext, dst_gs, ze)
  h0, h1, h2, h3 = _tc1(
      x_game, agg0, agg1, W_self_g, W_nbr_g, b_g.reshape(1, HID))
  g0, g1, g2, g3 = sc_stage2(h0, h1, h2, h3, src_gs, dst_gs, zx)
  z_state = _tc2(
      x_state, g0, g1, g2, g3, e0, e1, W_edge, W_self_s, W_nbr_s,
      b_s.reshape(1, HID), W_lin, b_lin.reshape(1, OUT))
  return z_state, x_game


# speedup vs baseline: 4.6406x; 1.3581x over previous
import functools

import jax
import jax.numpy as jnp
from jax import lax
from jax.experimental import pallas as pl
from jax.experimental.pallas import tpu as pltpu
from jax.experimental.pallas import tpu_sc as plsc

NG = 10000
NS_NODES = 10000
EGG = 160000
EGS = 160000
D = 256
HID = 512
OUT = 256
DE = 16
DE_PAD = 128

NC, NSUB = 2, 16
CH = 128
NCHUNK = EGG // CH
ROWS_MAIN = 624
ROWS_LAST = NG - 15 * ROWS_MAIN


def _zero_own(w, zsrc, acc):
  @pl.when(w < NSUB - 1)
  def _():
    pltpu.sync_copy(zsrc.at[pl.ds(0, ROWS_MAIN)],
                    acc.at[pl.ds(w * ROWS_MAIN, ROWS_MAIN)])
  @pl.when(w == NSUB - 1)
  def _():
    pltpu.sync_copy(zsrc, acc.at[pl.ds(15 * ROWS_MAIN, ROWS_LAST)])


def _copy_own(w, acc, out):
  @pl.when(w < NSUB - 1)
  def _():
    pltpu.sync_copy(acc.at[pl.ds(w * ROWS_MAIN, ROWS_MAIN)],
                    out.at[pl.ds(w * ROWS_MAIN, ROWS_MAIN)])
  @pl.when(w == NSUB - 1)
  def _():
    pltpu.sync_copy(acc.at[pl.ds(15 * ROWS_MAIN, ROWS_LAST)],
                    out.at[pl.ds(15 * ROWS_MAIN, ROWS_LAST)])

_f32 = jnp.float32


def _edge_pass(src_hbm, dst_hbm, table, acc,
               sa, da, sb, db, rows_v, sg, sp0, sp1, w):
  def pref(q, s_v, d_v, sem):
    pltpu.async_copy(src_hbm.at[pl.ds(q * CH, CH)], s_v, sem)
    pltpu.async_copy(dst_hbm.at[pl.ds(q * CH, CH)], d_v, sem)

  def drain(q, s_v, d_v, sem):
    pltpu.make_async_copy(src_hbm.at[pl.ds(q * CH, CH)], s_v, sem).wait()
    pltpu.make_async_copy(dst_hbm.at[pl.ds(q * CH, CH)], d_v, sem).wait()

  def slot(q, qn, s_v, d_v, sem, sn_v, dn_v, semn):
    @pl.when(q < NCHUNK)
    def _():
      drain(q, s_v, d_v, sem)
      @pl.when(qn < NCHUNK)
      def _():
        pref(qn, sn_v, dn_v, semn)
      pltpu.async_copy(table.at[s_v], rows_v, sg).wait()
      pltpu.sync_copy(rows_v, acc.at[d_v], add=True)

  pref(w, sa, da, sp0)
  def body(j, carry):
    q0 = (2 * j) * NSUB + w
    q1 = q0 + NSUB
    slot(q0, q1, sa, da, sp0, sb, db, sp1)
    slot(q1, q1 + NSUB, sb, db, sp1, sa, da, sp0)
    return carry
  lax.fori_loop(0, (NCHUNK + 2 * NSUB - 1) // (2 * NSUB), body, 0)


@functools.cache
def _sc_kernels():
  mesh = plsc.VectorSubcoreMesh(
      core_axis_name="c", subcore_axis_name="s",
      num_cores=NC, num_subcores=NSUB)

  @functools.partial(
      pl.kernel,
      out_type=(
          jax.ShapeDtypeStruct((NG, 128), _f32),
          jax.ShapeDtypeStruct((NG, 128), _f32),
      ),
      mesh=mesh,
      scratch_types=[
          pltpu.VMEM((CH,), jnp.int32),
          pltpu.VMEM((CH,), jnp.int32),
          pltpu.VMEM((CH,), jnp.int32),
          pltpu.VMEM((CH,), jnp.int32),
          pltpu.VMEM((CH, 128), _f32),
          pltpu.VMEM_SHARED((NG, 128), _f32),
          pltpu.SemaphoreType.DMA,
          pltpu.SemaphoreType.DMA,
          pltpu.SemaphoreType.DMA,
      ],
  )
  def _sc_stage1(xg0, xg1, src_gg, dst_gg, zx,
                 agg0, agg1,
                 sa, da, sb, db, rows_v, acc_x, sg, sp0, sp1):
    c = lax.axis_index("c")
    w = lax.axis_index("s")

    _zero_own(w, zx, acc_x)
    plsc.subcore_barrier()

    @pl.when(c == 0)
    def _():
      _edge_pass(src_gg, dst_gg, xg0, acc_x,
                 sa, da, sb, db, rows_v, sg, sp0, sp1, w)
    @pl.when(c == 1)
    def _():
      _edge_pass(src_gg, dst_gg, xg1, acc_x,
                 sa, da, sb, db, rows_v, sg, sp0, sp1, w)

    plsc.subcore_barrier()

    @pl.when(c == 0)
    def _():
      _copy_own(w, acc_x, agg0)
    @pl.when(c == 1)
    def _():
      _copy_own(w, acc_x, agg1)

  @functools.partial(
      pl.kernel,
      out_type=(
          jax.ShapeDtypeStruct((NS_NODES, DE_PAD), _f32),
          jax.ShapeDtypeStruct((NS_NODES, DE_PAD), _f32),
      ),
      mesh=mesh,
      scratch_types=[
          pltpu.VMEM((CH,), jnp.int32),
          pltpu.VMEM((CH,), jnp.int32),
          pltpu.VMEM((CH, DE_PAD), _f32),
          pltpu.VMEM((CH, DE_PAD), _f32),
          pltpu.VMEM_SHARED((NS_NODES, DE_PAD), _f32),
          pltpu.SemaphoreType.DMA,
          pltpu.SemaphoreType.DMA,
      ],
  )
  def _sc_edges(eext, dst_gs, ze, e0, e1,
                db0, db1, eb0, eb1, acc_e, se0, se1):
    c = lax.axis_index("c")
    w = lax.axis_index("s")

    _zero_own(w, ze, acc_e)
    plsc.subcore_barrier()

    half = NCHUNK // 2

    def pref(t, d_v, e_v, sem):
      off = (c * half + t) * CH
      pltpu.async_copy(dst_gs.at[pl.ds(off, CH)], d_v, sem)
      pltpu.async_copy(eext.at[pl.ds(off, CH)], e_v, sem)

    def slot(t, tn, d_v, e_v, sem, dn_v, en_v, semn):
      @pl.when(t < half)
      def _():
        off = (c * half + t) * CH
        pltpu.make_async_copy(dst_gs.at[pl.ds(off, CH)], d_v, sem).wait()
        pltpu.make_async_copy(eext.at[pl.ds(off, CH)], e_v, sem).wait()
        @pl.when(tn < half)
        def _():
          pref(tn, dn_v, en_v, semn)
        pltpu.sync_copy(e_v, acc_e.at[d_v], add=True)

    pref(w, db0, eb0, se0)
    def ebody(j, carry):
      t0 = (2 * j) * NSUB + w
      t1 = t0 + NSUB
      slot(t0, t1, db0, eb0, se0, db1, eb1, se1)
      slot(t1, t1 + NSUB, db1, eb1, se1, db0, eb0, se0)
      return carry
    lax.fori_loop(0, (half + 2 * NSUB - 1) // (2 * NSUB), ebody, 0)

    plsc.subcore_barrier()

    @pl.when(c == 0)
    def _():
      _copy_own(w, acc_e, e0)
    @pl.when(c == 1)
    def _():
      _copy_own(w, acc_e, e1)

  @functools.partial(
      pl.kernel,
      out_type=tuple(
          jax.ShapeDtypeStruct((NS_NODES, 128), _f32) for _ in range(4)),
      mesh=mesh,
      scratch_types=[
          pltpu.VMEM((CH,), jnp.int32),
          pltpu.VMEM((CH,), jnp.int32),
          pltpu.VMEM((CH,), jnp.int32),
          pltpu.VMEM((CH,), jnp.int32),
          pltpu.VMEM((CH, 128), _f32),
          pltpu.VMEM_SHARED((NS_NODES, 128), _f32),
          pltpu.SemaphoreType.DMA,
          pltpu.SemaphoreType.DMA,
          pltpu.SemaphoreType.DMA,
      ],
  )
  def _sc_stage2(h0, h1, h2, h3, src_gs, dst_gs, zx,
                 g0, g1, g2, g3,
                 sa, da, sb, db, rows_v, acc, sg, sp0, sp1):
    c = lax.axis_index("c")
    w = lax.axis_index("s")

    def one_pass(tbl, out):
      _zero_own(w, zx, acc)
      plsc.subcore_barrier()
      _edge_pass(src_gs, dst_gs, tbl, acc,
                 sa, da, sb, db, rows_v, sg, sp0, sp1, w)
      plsc.subcore_barrier()
      _copy_own(w, acc, out)
      plsc.subcore_barrier()

    @pl.when(c == 0)
    def _():
      one_pass(h0, g0)
      one_pass(h1, g1)
    @pl.when(c == 1)
    def _():
      one_pass(h2, g2)
      one_pass(h3, g3)

  return _sc_stage1, _sc_edges, _sc_stage2


_RB = 1000


def _tc1_body(x_ref, a0_ref, a1_ref, ws_ref, wn_ref, b_ref,
              o0, o1, o2, o3):
  h = jnp.dot(x_ref[...], ws_ref[...], preferred_element_type=_f32)
  h += jnp.dot(a0_ref[...], wn_ref[0:128, :], preferred_element_type=_f32)
  h += jnp.dot(a1_ref[...], wn_ref[128:256, :], preferred_element_type=_f32)
  h = jnp.maximum(h + b_ref[...], 0.0)
  o0[...] = h[:, 0:128]
  o1[...] = h[:, 128:256]
  o2[...] = h[:, 256:384]
  o3[...] = h[:, 384:512]


def _tc1(x_game, agg0, agg1, w_self, w_nbr, b):
  grid = (NG // _RB,)
  return pl.pallas_call(
      _tc1_body,
      grid=grid,
      in_specs=[
          pl.BlockSpec((_RB, D), lambda i: (i, 0)),
          pl.BlockSpec((_RB, 128), lambda i: (i, 0)),
          pl.BlockSpec((_RB, 128), lambda i: (i, 0)),
          pl.BlockSpec((D, HID), lambda i: (0, 0)),
          pl.BlockSpec((D, HID), lambda i: (0, 0)),
          pl.BlockSpec((1, HID), lambda i: (0, 0)),
      ],
      out_specs=tuple(
          pl.BlockSpec((_RB, 128), lambda i: (i, 0)) for _ in range(4)),
      out_shape=tuple(
          jax.ShapeDtypeStruct((NG, 128), _f32) for _ in range(4)),
  )(x_game, agg0, agg1, w_self, w_nbr, b)


def _tc2_body(xs_ref, g0, g1, g2, g3, e0, e1, we_ref, wss_ref, wns_ref,
              bs_ref, wl_ref, bl_ref, out_ref):
  ee = e0[...] + e1[...]
  cnt = jnp.maximum(ee[:, DE:DE + 1], 1.0)
  hg = jnp.concatenate([g0[...], g1[...], g2[...], g3[...]], axis=1)
  pre = hg + jnp.dot(ee[:, 0:DE], we_ref[...], preferred_element_type=_f32)
  agg_s = jnp.dot(pre, wns_ref[...], preferred_element_type=_f32) / cnt
  hs = jnp.dot(xs_ref[...], wss_ref[...], preferred_element_type=_f32)
  hs = jnp.maximum(hs + agg_s + bs_ref[...], 0.0)
  out_ref[...] = (
      jnp.dot(hs, wl_ref[...], preferred_element_type=_f32) + bl_ref[...])


def _tc2(x_state, g0, g1, g2, g3, e0, e1, w_edge, w_self_s, w_nbr_s, b_s,
         w_lin, b_lin):
  grid = (NS_NODES // _RB,)
  return pl.pallas_call(
      _tc2_body,
      grid=grid,
      in_specs=[
          pl.BlockSpec((_RB, D), lambda i: (i, 0)),
          pl.BlockSpec((_RB, 128), lambda i: (i, 0)),
          pl.BlockSpec((_RB, 128), lambda i: (i, 0)),
          pl.BlockSpec((_RB, 128), lambda i: (i, 0)),
          pl.BlockSpec((_RB, 128), lambda i: (i, 0)),
          pl.BlockSpec((_RB, DE_PAD), lambda i: (i, 0)),
          pl.BlockSpec((_RB, DE_PAD), lambda i: (i, 0)),
          pl.BlockSpec((DE, HID), lambda i: (0, 0)),
          pl.BlockSpec((D, HID), lambda i: (0, 0)),
          pl.BlockSpec((HID, HID), lambda i: (0, 0)),
          pl.BlockSpec((1, HID), lambda i: (0, 0)),
          pl.BlockSpec((HID, OUT), lambda i: (0, 0)),
          pl.BlockSpec((1, OUT), lambda i: (0, 0)),
      ],
      out_specs=pl.BlockSpec((_RB, OUT), lambda i: (i, 0)),
      out_shape=jax.ShapeDtypeStruct((NS_NODES, OUT), _f32),
  )(x_state, g0, g1, g2, g3, e0, e1, w_edge, w_self_s, w_nbr_s, b_s,
    w_lin, b_lin)


def kernel(x_game, x_state, edge_index_gg, edge_index_gs, edge_attr,
           W_self_g, W_nbr_g, b_g, W_edge, W_self_s, W_nbr_s, b_s,
           W_lin, b_lin):
  xg0 = x_game[:, 0:128]
  xg1 = x_game[:, 128:256]
  src_gg = edge_index_gg[0]
  dst_gg = edge_index_gg[1]
  src_gs = edge_index_gs[0]
  dst_gs = edge_index_gs[1]
  eext = jnp.concatenate(
      [edge_attr,
       jnp.ones((EGS, 1), _f32),
       jnp.zeros((EGS, DE_PAD - DE - 1), _f32)], axis=1)
  zx = jnp.zeros((ROWS_LAST, 128), _f32)
  ze = jnp.zeros((ROWS_LAST, DE_PAD), _f32)

  sc_stage1, sc_edges, sc_stage2 = _sc_kernels()
  agg0, agg1 = sc_stage1(xg0, xg1, src_gg, dst_gg, zx)
  e0, e1 = sc_edges(eext, dst_gs, ze)
  h0, h1, h2, h3 = _tc1(
      x_game, agg0, agg1, W_self_g, W_nbr_g, b_g.reshape(1, HID))
  g0, g1, g2, g3 = sc_stage2(h0, h1, h2, h3, src_gs, dst_gs, zx)
  z_state = _tc2(
      x_state, g0, g1, g2, g3, e0, e1, W_edge, W_self_s, W_nbr_s,
      b_s.reshape(1, HID), W_lin, b_lin.reshape(1, OUT))
  return z_state, x_game


# speedup vs baseline: 5.7145x; 1.2314x over previous
import functools

import jax
import jax.numpy as jnp
from jax import lax
from jax.experimental import pallas as pl
from jax.experimental.pallas import tpu as pltpu
from jax.experimental.pallas import tpu_sc as plsc

NG = 10000
NS_NODES = 10000
EGG = 160000
EGS = 160000
D = 256
HID = 512
OUT = 256
DE = 16
DE_PAD = 128

NC, NSUB = 2, 16
CH = 128
NCHUNK = EGG // CH
ROWS_MAIN = 624
ROWS_LAST = NG - 15 * ROWS_MAIN


def _zero_own(w, zsrc, acc):
  @pl.when(w < NSUB - 1)
  def _():
    pltpu.sync_copy(zsrc.at[pl.ds(0, ROWS_MAIN)],
                    acc.at[pl.ds(w * ROWS_MAIN, ROWS_MAIN)])
  @pl.when(w == NSUB - 1)
  def _():
    pltpu.sync_copy(zsrc, acc.at[pl.ds(15 * ROWS_MAIN, ROWS_LAST)])


def _copy_own(w, acc, out):
  @pl.when(w < NSUB - 1)
  def _():
    pltpu.sync_copy(acc.at[pl.ds(w * ROWS_MAIN, ROWS_MAIN)],
                    out.at[pl.ds(w * ROWS_MAIN, ROWS_MAIN)])
  @pl.when(w == NSUB - 1)
  def _():
    pltpu.sync_copy(acc.at[pl.ds(15 * ROWS_MAIN, ROWS_LAST)],
                    out.at[pl.ds(15 * ROWS_MAIN, ROWS_LAST)])

_f32 = jnp.float32


def _edge_pass(src_hbm, dst_hbm, table, acc,
               s0, s1, d0, d1, d2, d3, r0, r1,
               sg0, sg1, ss0, ss1, sp0, sp1, w):
  svs = (s0, s1)
  dvs = (d0, d1, d2, d3)
  rvs = (r0, r1)
  sgs = (sg0, sg1)
  sss = (ss0, ss1)
  sps = (sp0, sp1)

  def pref(q, k, sem):
    pltpu.async_copy(src_hbm.at[pl.ds(q * CH, CH)], svs[k % 2], sem)
    pltpu.async_copy(dst_hbm.at[pl.ds(q * CH, CH)], dvs[k % 4], sem)

  def drain_idx(q, k, sem):
    pltpu.make_async_copy(
        src_hbm.at[pl.ds(q * CH, CH)], svs[k % 2], sem).wait()
    pltpu.make_async_copy(
        dst_hbm.at[pl.ds(q * CH, CH)], dvs[k % 4], sem).wait()

  def slot(j, i, k):
    q = i * NSUB + w
    @pl.when(q < NCHUNK)
    def _():
      drain_idx(q, k, sps[k % 2])
      @pl.when(q + NSUB < NCHUNK)
      def _():
        pref(q + NSUB, k + 1, sps[(k + 1) % 2])
      if k >= 2:
        pltpu.make_async_copy(rvs[k % 2], acc.at[dvs[(k - 2) % 4]],
                              sss[k % 2]).wait()
      else:
        @pl.when(j > 0)
        def _():
          pltpu.make_async_copy(rvs[k % 2], acc.at[dvs[(k + 2) % 4]],
                                sss[k % 2]).wait()
      pltpu.async_copy(table.at[svs[k % 2]], rvs[k % 2], sgs[k % 2]).wait()
      pltpu.async_copy(rvs[k % 2], acc.at[dvs[k % 4]], sss[k % 2],
                       add=True)

  pref(w, 0, sps[0])
  def body(j, carry):
    for k in range(4):
      slot(j, 4 * j + k, k)
    return carry
  lax.fori_loop(0, (NCHUNK + 4 * NSUB - 1) // (4 * NSUB), body, 0)
  pltpu.make_async_copy(r0, acc.at[d0], ss0).wait()
  pltpu.make_async_copy(r1, acc.at[d1], ss1).wait()


@functools.cache
def _sc_kernels():
  mesh = plsc.VectorSubcoreMesh(
      core_axis_name="c", subcore_axis_name="s",
      num_cores=NC, num_subcores=NSUB)

  @functools.partial(
      pl.kernel,
      out_type=(
          jax.ShapeDtypeStruct((NG, 128), _f32),
          jax.ShapeDtypeStruct((NG, 128), _f32),
      ),
      mesh=mesh,
      scratch_types=[
          pltpu.VMEM((CH,), jnp.int32),
          pltpu.VMEM((CH,), jnp.int32),
          pltpu.VMEM((CH,), jnp.int32),
          pltpu.VMEM((CH,), jnp.int32),
          pltpu.VMEM((CH,), jnp.int32),
          pltpu.VMEM((CH,), jnp.int32),
          pltpu.VMEM((CH, 128), _f32),
          pltpu.VMEM((CH, 128), _f32),
          pltpu.VMEM_SHARED((NG, 128), _f32),
          pltpu.SemaphoreType.DMA,
          pltpu.SemaphoreType.DMA,
          pltpu.SemaphoreType.DMA,
          pltpu.SemaphoreType.DMA,
          pltpu.SemaphoreType.DMA,
          pltpu.SemaphoreType.DMA,
      ],
  )
  def _sc_stage1(xg0, xg1, src_gg, dst_gg, zx,
                 agg0, agg1,
                 s0, s1, d0, d1, d2, d3, r0, r1, acc_x,
                 sg0, sg1, ss0, ss1, sp0, sp1):
    c = lax.axis_index("c")
    w = lax.axis_index("s")

    _zero_own(w, zx, acc_x)
    plsc.subcore_barrier()

    @pl.when(c == 0)
    def _():
      _edge_pass(src_gg, dst_gg, xg0, acc_x,
                 s0, s1, d0, d1, d2, d3, r0, r1,
                 sg0, sg1, ss0, ss1, sp0, sp1, w)
    @pl.when(c == 1)
    def _():
      _edge_pass(src_gg, dst_gg, xg1, acc_x,
                 s0, s1, d0, d1, d2, d3, r0, r1,
                 sg0, sg1, ss0, ss1, sp0, sp1, w)

    plsc.subcore_barrier()

    @pl.when(c == 0)
    def _():
      _copy_own(w, acc_x, agg0)
    @pl.when(c == 1)
    def _():
      _copy_own(w, acc_x, agg1)

  @functools.partial(
      pl.kernel,
      out_type=(
          jax.ShapeDtypeStruct((NS_NODES, DE_PAD), _f32),
          jax.ShapeDtypeStruct((NS_NODES, DE_PAD), _f32),
      ),
      mesh=mesh,
      scratch_types=[
          pltpu.VMEM((CH,), jnp.int32),
          pltpu.VMEM((CH,), jnp.int32),
          pltpu.VMEM((CH, DE_PAD), _f32),
          pltpu.VMEM((CH, DE_PAD), _f32),
          pltpu.VMEM_SHARED((NS_NODES, DE_PAD), _f32),
          pltpu.SemaphoreType.DMA,
          pltpu.SemaphoreType.DMA,
      ],
  )
  def _sc_edges(eext, dst_gs, ze, e0, e1,
                db0, db1, eb0, eb1, acc_e, se0, se1):
    c = lax.axis_index("c")
    w = lax.axis_index("s")

    _zero_own(w, ze, acc_e)
    plsc.subcore_barrier()

    half = NCHUNK // 2

    def pref(t, d_v, e_v, sem):
      off = (c * half + t) * CH
      pltpu.async_copy(dst_gs.at[pl.ds(off, CH)], d_v, sem)
      pltpu.async_copy(eext.at[pl.ds(off, CH)], e_v, sem)

    def slot(t, tn, d_v, e_v, sem, dn_v, en_v, semn):
      @pl.when(t < half)
      def _():
        off = (c * half + t) * CH
        pltpu.make_async_copy(dst_gs.at[pl.ds(off, CH)], d_v, sem).wait()
        pltpu.make_async_copy(eext.at[pl.ds(off, CH)], e_v, sem).wait()
        @pl.when(tn < half)
        def _():
          pref(tn, dn_v, en_v, semn)
        pltpu.sync_copy(e_v, acc_e.at[d_v], add=True)

    pref(w, db0, eb0, se0)
    def ebody(j, carry):
      t0 = (2 * j) * NSUB + w
      t1 = t0 + NSUB
      slot(t0, t1, db0, eb0, se0, db1, eb1, se1)
      slot(t1, t1 + NSUB, db1, eb1, se1, db0, eb0, se0)
      return carry
    lax.fori_loop(0, (half + 2 * NSUB - 1) // (2 * NSUB), ebody, 0)

    plsc.subcore_barrier()

    @pl.when(c == 0)
    def _():
      _copy_own(w, acc_e, e0)
    @pl.when(c == 1)
    def _():
      _copy_own(w, acc_e, e1)

  @functools.partial(
      pl.kernel,
      out_type=tuple(
          jax.ShapeDtypeStruct((NS_NODES, 128), _f32) for _ in range(4)),
      mesh=mesh,
      scratch_types=[
          pltpu.VMEM((CH,), jnp.int32),
          pltpu.VMEM((CH,), jnp.int32),
          pltpu.VMEM((CH,), jnp.int32),
          pltpu.VMEM((CH,), jnp.int32),
          pltpu.VMEM((CH,), jnp.int32),
          pltpu.VMEM((CH,), jnp.int32),
          pltpu.VMEM((CH, 128), _f32),
          pltpu.VMEM((CH, 128), _f32),
          pltpu.VMEM_SHARED((NS_NODES, 128), _f32),
          pltpu.SemaphoreType.DMA,
          pltpu.SemaphoreType.DMA,
          pltpu.SemaphoreType.DMA,
          pltpu.SemaphoreType.DMA,
          pltpu.SemaphoreType.DMA,
          pltpu.SemaphoreType.DMA,
      ],
  )
  def _sc_stage2(h0, h1, h2, h3, src_gs, dst_gs, zx,
                 g0, g1, g2, g3,
                 s0, s1, d0, d1, d2, d3, r0, r1, acc,
                 sg0, sg1, ss0, ss1, sp0, sp1):
    c = lax.axis_index("c")
    w = lax.axis_index("s")

    def one_pass(tbl, out):
      _zero_own(w, zx, acc)
      plsc.subcore_barrier()
      _edge_pass(src_gs, dst_gs, tbl, acc,
                 s0, s1, d0, d1, d2, d3, r0, r1,
                 sg0, sg1, ss0, ss1, sp0, sp1, w)
      plsc.subcore_barrier()
      _copy_own(w, acc, out)
      plsc.subcore_barrier()

    @pl.when(c == 0)
    def _():
      one_pass(h0, g0)
      one_pass(h1, g1)
    @pl.when(c == 1)
    def _():
      one_pass(h2, g2)
      one_pass(h3, g3)

  return _sc_stage1, _sc_edges, _sc_stage2


_RB = 1000


def _tc1_body(x_ref, a0_ref, a1_ref, ws_ref, wn_ref, b_ref,
              o0, o1, o2, o3):
  h = jnp.dot(x_ref[...], ws_ref[...], preferred_element_type=_f32)
  h += jnp.dot(a0_ref[...], wn_ref[0:128, :], preferred_element_type=_f32)
  h += jnp.dot(a1_ref[...], wn_ref[128:256, :], preferred_element_type=_f32)
  h = jnp.maximum(h + b_ref[...], 0.0)
  o0[...] = h[:, 0:128]
  o1[...] = h[:, 128:256]
  o2[...] = h[:, 256:384]
  o3[...] = h[:, 384:512]


def _tc1(x_game, agg0, agg1, w_self, w_nbr, b):
  grid = (NG // _RB,)
  return pl.pallas_call(
      _tc1_body,
      grid=grid,
      in_specs=[
          pl.BlockSpec((_RB, D), lambda i: (i, 0)),
          pl.BlockSpec((_RB, 128), lambda i: (i, 0)),
          pl.BlockSpec((_RB, 128), lambda i: (i, 0)),
          pl.BlockSpec((D, HID), lambda i: (0, 0)),
          pl.BlockSpec((D, HID), lambda i: (0, 0)),
          pl.BlockSpec((1, HID), lambda i: (0, 0)),
      ],
      out_specs=tuple(
          pl.BlockSpec((_RB, 128), lambda i: (i, 0)) for _ in range(4)),
      out_shape=tuple(
          jax.ShapeDtypeStruct((NG, 128), _f32) for _ in range(4)),
  )(x_game, agg0, agg1, w_self, w_nbr, b)


def _tc2_body(xs_ref, g0, g1, g2, g3, e0, e1, we_ref, wss_ref, wns_ref,
              bs_ref, wl_ref, bl_ref, out_ref):
  ee = e0[...] + e1[...]
  cnt = jnp.maximum(ee[:, DE:DE + 1], 1.0)
  hg = jnp.concatenate([g0[...], g1[...], g2[...], g3[...]], axis=1)
  pre = hg + jnp.dot(ee[:, 0:DE], we_ref[...], preferred_element_type=_f32)
  agg_s = jnp.dot(pre, wns_ref[...], preferred_element_type=_f32) / cnt
  hs = jnp.dot(xs_ref[...], wss_ref[...], preferred_element_type=_f32)
  hs = jnp.maximum(hs + agg_s + bs_ref[...], 0.0)
  out_ref[...] = (
      jnp.dot(hs, wl_ref[...], preferred_element_type=_f32) + bl_ref[...])


def _tc2(x_state, g0, g1, g2, g3, e0, e1, w_edge, w_self_s, w_nbr_s, b_s,
         w_lin, b_lin):
  grid = (NS_NODES // _RB,)
  return pl.pallas_call(
      _tc2_body,
      grid=grid,
      in_specs=[
          pl.BlockSpec((_RB, D), lambda i: (i, 0)),
          pl.BlockSpec((_RB, 128), lambda i: (i, 0)),
          pl.BlockSpec((_RB, 128), lambda i: (i, 0)),
          pl.BlockSpec((_RB, 128), lambda i: (i, 0)),
          pl.BlockSpec((_RB, 128), lambda i: (i, 0)),
          pl.BlockSpec((_RB, DE_PAD), lambda i: (i, 0)),
          pl.BlockSpec((_RB, DE_PAD), lambda i: (i, 0)),
          pl.BlockSpec((DE, HID), lambda i: (0, 0)),
          pl.BlockSpec((D, HID), lambda i: (0, 0)),
          pl.BlockSpec((HID, HID), lambda i: (0, 0)),
          pl.BlockSpec((1, HID), lambda i: (0, 0)),
          pl.BlockSpec((HID, OUT), lambda i: (0, 0)),
          pl.BlockSpec((1, OUT), lambda i: (0, 0)),
      ],
      out_specs=pl.BlockSpec((_RB, OUT), lambda i: (i, 0)),
      out_shape=jax.ShapeDtypeStruct((NS_NODES, OUT), _f32),
  )(x_state, g0, g1, g2, g3, e0, e1, w_edge, w_self_s, w_nbr_s, b_s,
    w_lin, b_lin)


def kernel(x_game, x_state, edge_index_gg, edge_index_gs, edge_attr,
           W_self_g, W_nbr_g, b_g, W_edge, W_self_s, W_nbr_s, b_s,
           W_lin, b_lin):
  xg0 = x_game[:, 0:128]
  xg1 = x_game[:, 128:256]
  src_gg = edge_index_gg[0]
  dst_gg = edge_index_gg[1]
  src_gs = edge_index_gs[0]
  dst_gs = edge_index_gs[1]
  eext = jnp.concatenate(
      [edge_attr,
       jnp.ones((EGS, 1), _f32),
       jnp.zeros((EGS, DE_PAD - DE - 1), _f32)], axis=1)
  zx = jnp.zeros((ROWS_LAST, 128), _f32)
  ze = jnp.zeros((ROWS_LAST, DE_PAD), _f32)

  sc_stage1, sc_edges, sc_stage2 = _sc_kernels()
  agg0, agg1 = sc_stage1(xg0, xg1, src_gg, dst_gg, zx)
  e0, e1 = sc_edges(eext, dst_gs, ze)
  h0, h1, h2, h3 = _tc1(
      x_game, agg0, agg1, W_self_g, W_nbr_g, b_g.reshape(1, HID))
  g0, g1, g2, g3 = sc_stage2(h0, h1, h2, h3, src_gs, dst_gs, zx)
  z_state = _tc2(
      x_state, g0, g1, g2, g3, e0, e1, W_edge, W_self_s, W_nbr_s,
      b_s.reshape(1, HID), W_lin, b_lin.reshape(1, OUT))
  return z_state, x_game
